# baseline (speedup 1.0000x reference)
import functools

import jax
import jax.numpy as jnp
import numpy as np
from jax import lax
from jax.experimental import pallas as pl
from jax.experimental.pallas import tpu as pltpu
from jax.experimental.pallas import tpu_sc as plsc

VOCAB = 100000
EMB = 256
HID = 2048
OUT = 4096
B = 4096
L = 200

NC = 2
NS = 16
NW = NC * NS
SEG_PER_W = B // NW
LANES = 16
NCH = EMB // LANES

CH0, CH1 = 128, 72


def _pool_body(tok_hbm, table_hbm, out_hbm, idx_v, rows_a, rows_b, accst,
               sem_a, sem_b):
    wid = lax.axis_index("s") * NC + lax.axis_index("c")
    seg0 = wid * SEG_PER_W

    pltpu.sync_copy(tok_hbm.at[pl.ds(seg0 * L, SEG_PER_W * L)], idx_v)

    def issue(seg, rows, sem):
        off = seg * L
        pltpu.async_copy(table_hbm.at[idx_v.at[pl.ds(off, CH0)]],
                         rows.at[pl.ds(0, CH0)], sem)
        pltpu.async_copy(table_hbm.at[idx_v.at[pl.ds(off + CH0, CH1)]],
                         rows.at[pl.ds(CH0, CH1)], sem)

    def wait(seg, rows, sem):
        off = seg * L
        pltpu.make_async_copy(table_hbm.at[idx_v.at[pl.ds(off, CH0)]],
                              rows.at[pl.ds(0, CH0)], sem).wait()
        pltpu.make_async_copy(table_hbm.at[idx_v.at[pl.ds(off + CH0, CH1)]],
                              rows.at[pl.ds(CH0, CH1)], sem).wait()

    def acc_store(seg, rows):
        def body8(r, acc):
            acc = list(acc)
            for u in range(8):
                for c in range(NCH // 2):
                    w = rows[r * 8 + u, pl.ds(c * LANES, LANES)]
                    a = lax.bitcast_convert_type(
                        lax.shift_left(w, 16), jnp.float32)
                    b = lax.bitcast_convert_type(
                        lax.bitwise_and(w, jnp.int32(-65536)), jnp.float32)
                    acc[2 * c] = acc[2 * c] + a
                    acc[2 * c + 1] = acc[2 * c + 1] + b
            return tuple(acc)

        acc = tuple(jnp.zeros((LANES,), jnp.float32) for _ in range(NCH))
        acc = lax.fori_loop(0, L // 8, body8, acc)
        inv = jnp.float32(1.0 / L)
        for j in range(NCH):
            accst[0, pl.ds(j * LANES, LANES)] = acc[j] * inv
        pltpu.sync_copy(accst, out_hbm.at[pl.ds(seg0 + seg, 1)])

    issue(0, rows_a, sem_a)
    issue(1, rows_b, sem_b)

    def pair(i, carry):
        sa = i * 2
        wait(sa, rows_a, sem_a)
        acc_store(sa, rows_a)
        issue(sa + 2, rows_a, sem_a)
        wait(sa + 1, rows_b, sem_b)
        acc_store(sa + 1, rows_b)
        issue(sa + 3, rows_b, sem_b)
        return carry

    lax.fori_loop(0, SEG_PER_W // 2 - 1, pair, 0)
    last = SEG_PER_W - 2
    wait(last, rows_a, sem_a)
    acc_store(last, rows_a)
    wait(last + 1, rows_b, sem_b)
    acc_store(last + 1, rows_b)


@functools.cache
def _pool():
    return functools.partial(
        pl.kernel,
        out_type=jax.ShapeDtypeStruct((B, EMB), jnp.float32),
        mesh=plsc.VectorSubcoreMesh(core_axis_name="c", subcore_axis_name="s",
                                    num_cores=NC, num_subcores=NS),
        scratch_types=[
            pltpu.VMEM((SEG_PER_W * L,), jnp.int32),
            pltpu.VMEM((L, EMB // 2), jnp.int32),
            pltpu.VMEM((L, EMB // 2), jnp.int32),
            pltpu.VMEM((1, EMB), jnp.float32),
            pltpu.SemaphoreType.DMA,
            pltpu.SemaphoreType.DMA,
        ],
    )(_pool_body)


def _mlp_body(x_ref, w1_ref, b1_ref, g_ref, bt_ref, w2_ref, b2_ref, o_ref):
    x = x_ref[...]
    h = jnp.dot(x, w1_ref[...], preferred_element_type=jnp.float32) + b1_ref[...]
    h = 0.5 * h * (1.0 + lax.erf(h * jnp.float32(0.7071067811865476)))
    mu = jnp.mean(h, axis=-1, keepdims=True)
    hc = h - mu
    var = jnp.mean(hc * hc, axis=-1, keepdims=True)
    h = hc * lax.rsqrt(var + 1e-5)
    h = h * g_ref[...] + bt_ref[...]
    out = jnp.dot(h, w2_ref[...], preferred_element_type=jnp.float32) + b2_ref[...]
    n2 = jnp.sum(out * out, axis=-1, keepdims=True)
    o_ref[...] = out * lax.rsqrt(jnp.maximum(n2, 1e-24))


BLK = 256


def _mlp(pooled, W1, b1, gamma, beta, W2, b2):
    grid = (B // BLK,)
    return pl.pallas_call(
        _mlp_body,
        grid=grid,
        in_specs=[
            pl.BlockSpec((BLK, EMB), lambda i: (i, 0)),
            pl.BlockSpec((EMB, HID), lambda i: (0, 0)),
            pl.BlockSpec((1, HID), lambda i: (0, 0)),
            pl.BlockSpec((1, HID), lambda i: (0, 0)),
            pl.BlockSpec((1, HID), lambda i: (0, 0)),
            pl.BlockSpec((HID, OUT), lambda i: (0, 0)),
            pl.BlockSpec((1, OUT), lambda i: (0, 0)),
        ],
        out_specs=pl.BlockSpec((BLK, OUT), lambda i: (i, 0)),
        out_shape=jax.ShapeDtypeStruct((B, OUT), jnp.float32),
    )(pooled, W1, b1, gamma, beta, W2, b2)


_PERM = np.concatenate([
    np.concatenate([np.arange(c, c + 32, 2), np.arange(c + 1, c + 32, 2)])
    for c in range(0, EMB, 32)
])


def kernel(token_ids, table, W1, b1, gamma, beta, W2, b2):
    tok_flat = token_ids.reshape(-1).astype(jnp.int32)
    table_b = lax.bitcast_convert_type(
        table.astype(jnp.bfloat16).reshape(VOCAB, EMB // 2, 2),
        jnp.int32)
    pooled = _pool()(tok_flat, table_b)
    W1p = jnp.take(W1, jnp.asarray(_PERM), axis=0)
    return _mlp(pooled, W1p, b1.reshape(1, HID), gamma.reshape(1, HID),
                beta.reshape(1, HID), W2, b2.reshape(1, OUT))


# speedup vs baseline: 2.0761x; 2.0761x over previous
import functools

import jax
import jax.numpy as jnp
import numpy as np
from jax import lax
from jax.experimental import pallas as pl
from jax.experimental.pallas import tpu as pltpu
from jax.experimental.pallas import tpu_sc as plsc

VOCAB = 100000
EMB = 256
HID = 2048
OUT = 4096
B = 4096
L = 200

NC = 2
NS = 16
NW = NC * NS
SEG_PER_W = B // NW
LANES = 16
NCH = EMB // LANES

CH0, CH1 = 128, 72


def _pool_body(tok_hbm, table_hbm, out_hbm, idx_v, rows_a, rows_b, accst,
               sem_a, sem_b):
    wid = lax.axis_index("s") * NC + lax.axis_index("c")
    seg0 = wid * SEG_PER_W

    pltpu.sync_copy(tok_hbm.at[pl.ds(seg0 * L, SEG_PER_W * L)], idx_v)

    def issue(seg, rows, sem):
        off = seg * L
        pltpu.async_copy(table_hbm.at[idx_v.at[pl.ds(off, CH0)]],
                         rows.at[pl.ds(0, CH0)], sem)
        pltpu.async_copy(table_hbm.at[idx_v.at[pl.ds(off + CH0, CH1)]],
                         rows.at[pl.ds(CH0, CH1)], sem)

    def wait(seg, rows, sem):
        off = seg * L
        pltpu.make_async_copy(table_hbm.at[idx_v.at[pl.ds(off, CH0)]],
                              rows.at[pl.ds(0, CH0)], sem).wait()
        pltpu.make_async_copy(table_hbm.at[idx_v.at[pl.ds(off + CH0, CH1)]],
                              rows.at[pl.ds(CH0, CH1)], sem).wait()

    def acc_store(seg, rows):
        def body8(r, acc):
            acc = list(acc)
            for u in range(8):
                for c in range(NCH // 2):
                    w = rows[r * 8 + u, pl.ds(c * LANES, LANES)]
                    a = lax.bitcast_convert_type(
                        lax.shift_left(w, 16), jnp.float32)
                    b = lax.bitcast_convert_type(
                        lax.bitwise_and(w, jnp.int32(-65536)), jnp.float32)
                    acc[2 * c] = acc[2 * c] + a
                    acc[2 * c + 1] = acc[2 * c + 1] + b
            return tuple(acc)

        acc = tuple(jnp.zeros((LANES,), jnp.float32) for _ in range(NCH))
        acc = lax.fori_loop(0, L // 8, body8, acc)
        inv = jnp.float32(1.0 / L)
        for j in range(NCH):
            accst[0, pl.ds(j * LANES, LANES)] = acc[j] * inv
        pltpu.sync_copy(accst, out_hbm.at[pl.ds(seg0 + seg, 1)])

    issue(0, rows_a, sem_a)
    issue(1, rows_b, sem_b)

    def pair(i, carry):
        sa = i * 2
        wait(sa, rows_a, sem_a)
        acc_store(sa, rows_a)
        issue(sa + 2, rows_a, sem_a)
        wait(sa + 1, rows_b, sem_b)
        acc_store(sa + 1, rows_b)
        issue(sa + 3, rows_b, sem_b)
        return carry

    lax.fori_loop(0, SEG_PER_W // 2 - 1, pair, 0)
    last = SEG_PER_W - 2
    wait(last, rows_a, sem_a)
    acc_store(last, rows_a)
    wait(last + 1, rows_b, sem_b)
    acc_store(last + 1, rows_b)


@functools.cache
def _pool():
    return functools.partial(
        pl.kernel,
        out_type=jax.ShapeDtypeStruct((B, EMB), jnp.float32),
        mesh=plsc.VectorSubcoreMesh(core_axis_name="c", subcore_axis_name="s",
                                    num_cores=NC, num_subcores=NS),
        scratch_types=[
            pltpu.VMEM((SEG_PER_W * L,), jnp.int32),
            pltpu.VMEM((L, EMB // 2), jnp.int32),
            pltpu.VMEM((L, EMB // 2), jnp.int32),
            pltpu.VMEM((1, EMB), jnp.float32),
            pltpu.SemaphoreType.DMA,
            pltpu.SemaphoreType.DMA,
        ],
    )(_pool_body)


def _mlp_body(x_ref, w1_ref, b1_ref, g_ref, bt_ref, w2_ref, b2_ref, o_ref):
    x = x_ref[...]
    h = jnp.dot(x, w1_ref[...], preferred_element_type=jnp.float32) + b1_ref[...]
    h = 0.5 * h * (1.0 + lax.erf(h * jnp.float32(0.7071067811865476)))
    mu = jnp.mean(h, axis=-1, keepdims=True)
    hc = h - mu
    var = jnp.mean(hc * hc, axis=-1, keepdims=True)
    h = hc * lax.rsqrt(var + 1e-5)
    h = h * g_ref[...] + bt_ref[...]
    out = jnp.dot(h, w2_ref[...], preferred_element_type=jnp.float32) + b2_ref[...]
    n2 = jnp.sum(out * out, axis=-1, keepdims=True)
    o_ref[...] = out * lax.rsqrt(jnp.maximum(n2, 1e-24))


def _pack_body(t_ref, o_ref):
    x = t_ref[...]
    bits = lax.bitcast_convert_type(x, jnp.uint32)
    rnd = (bits + jnp.uint32(0x7FFF) + ((bits >> 16) & jnp.uint32(1))) >> 16
    lo = rnd[:, :EMB // 2]
    hi = rnd[:, EMB // 2:]
    o_ref[...] = lax.bitcast_convert_type(lo | (hi << 16), jnp.int32)


BLKV = 2000


def _pack(table):
    return pl.pallas_call(
        _pack_body,
        grid=(VOCAB // BLKV,),
        in_specs=[pl.BlockSpec((BLKV, EMB), lambda i: (i, 0))],
        out_specs=pl.BlockSpec((BLKV, EMB // 2), lambda i: (i, 0)),
        out_shape=jax.ShapeDtypeStruct((VOCAB, EMB // 2), jnp.int32),
    )(table)


BLK = 256


def _mlp(pooled, W1, b1, gamma, beta, W2, b2):
    grid = (B // BLK,)
    return pl.pallas_call(
        _mlp_body,
        grid=grid,
        in_specs=[
            pl.BlockSpec((BLK, EMB), lambda i: (i, 0)),
            pl.BlockSpec((EMB, HID), lambda i: (0, 0)),
            pl.BlockSpec((1, HID), lambda i: (0, 0)),
            pl.BlockSpec((1, HID), lambda i: (0, 0)),
            pl.BlockSpec((1, HID), lambda i: (0, 0)),
            pl.BlockSpec((HID, OUT), lambda i: (0, 0)),
            pl.BlockSpec((1, OUT), lambda i: (0, 0)),
        ],
        out_specs=pl.BlockSpec((BLK, OUT), lambda i: (i, 0)),
        out_shape=jax.ShapeDtypeStruct((B, OUT), jnp.float32),
    )(pooled, W1, b1, gamma, beta, W2, b2)


_PERM = np.concatenate([
    np.concatenate([np.arange(16 * c, 16 * c + 16),
                    np.arange(EMB // 2 + 16 * c, EMB // 2 + 16 * c + 16)])
    for c in range(EMB // 32)
])


def kernel(token_ids, table, W1, b1, gamma, beta, W2, b2):
    tok_flat = token_ids.reshape(-1).astype(jnp.int32)
    table_b = _pack(table)
    pooled = _pool()(tok_flat, table_b)
    W1p = jnp.take(W1, jnp.asarray(_PERM), axis=0)
    return _mlp(pooled, W1p, b1.reshape(1, HID), gamma.reshape(1, HID),
                beta.reshape(1, HID), W2, b2.reshape(1, OUT))
